# baseline (speedup 1.0000x reference)
import functools

import jax
import jax.numpy as jnp
from jax import lax
from jax.experimental import pallas as pl
from jax.experimental.pallas import tpu as pltpu
from jax.experimental.pallas import tpu_sc as plsc

NC = 2
NS = 16
NW = NC * NS
LANES = 16
CH = 16
NBUF = 4
PCH = 32


@functools.lru_cache(maxsize=None)
def _build(nb, seq, d):
    pw = seq // NW
    n_sub = PCH // CH
    n_h = pw // PCH
    n_chunks = n_h * nb * n_sub
    mesh = plsc.VectorSubcoreMesh(
        core_axis_name="c", subcore_axis_name="s",
        num_cores=NC, num_subcores=NS)

    @functools.partial(
        pl.kernel,
        out_type=jax.ShapeDtypeStruct((nb * seq, d), jnp.float32),
        mesh=mesh,
        scratch_types=(
            [pltpu.VMEM((pw,), jnp.int32) for _ in range(nb)]
            + [pltpu.VMEM((CH, d), jnp.float32) for _ in range(NBUF)]
            + [pltpu.VMEM((PCH, d), jnp.float32)]
            + [pltpu.SemaphoreType.DMA for _ in range(2 * NBUF)]
        ),
    )
    def emb(ids_hbm, wte_hbm, wpe_hbm, out_hbm, *refs):
        idx = refs[:nb]
        tok = refs[nb:nb + NBUF]
        pos_v = refs[nb + NBUF]
        sg = refs[nb + NBUF + 1:nb + NBUF + 1 + NBUF]
        ss = refs[nb + NBUF + 1 + NBUF:]
        wid = lax.axis_index("s") * NC + lax.axis_index("c")
        pbase = wid * pw

        def parts(k):
            h, r = divmod(k, nb * n_sub)
            b, j = divmod(r, n_sub)
            return h, b, j

        def add(p, j):
            def body(r, carry):
                for i in range(d // LANES):
                    sl = pl.ds(i * LANES, LANES)
                    tok[p][r, sl] = tok[p][r, sl] + pos_v[j * CH + r, sl]
                return carry
            lax.fori_loop(0, CH, body, 0)

        def start_gather(k):
            h, b, j = parts(k)
            p = k % NBUF
            return pltpu.async_copy(
                wte_hbm.at[idx[b].at[pl.ds(h * PCH + j * CH, CH)]],
                tok[p], sg[p])

        def finish(k):
            h, b, j = parts(k)
            p = k % NBUF
            g[p].wait()
            add(p, j)
            s[p] = pltpu.async_copy(
                tok[p],
                out_hbm.at[pl.ds(b * seq + pbase + h * PCH + j * CH, CH)],
                ss[p])

        g = [None] * NBUF
        s = [None] * NBUF

        for b in range(nb):
            pltpu.sync_copy(ids_hbm.at[b, pl.ds(pbase, pw)], idx[b])
        pltpu.sync_copy(wpe_hbm.at[pl.ds(pbase, PCH)], pos_v)

        g[0] = start_gather(0)
        g[1] = start_gather(1)
        half = n_chunks // n_h
        for k in range(2, n_chunks):
            p = k % NBUF
            if s[p] is not None:
                s[p].wait()
            g[p] = start_gather(k)
            if k >= half + 2 and (k - 2) % half == 0:
                hh = (k - 2) // half
                pltpu.sync_copy(wpe_hbm.at[pl.ds(pbase + hh * PCH, PCH)],
                                pos_v)
            finish(k - 2)
        finish(n_chunks - 2)
        finish(n_chunks - 1)
        for p in range(NBUF):
            if s[p] is not None:
                s[p].wait()

    return emb


def kernel(input_ids, wte, wpe):
    b, s = input_ids.shape
    d = wte.shape[1]
    emb = _build(b, s, d)
    out = emb(input_ids, wte, wpe)
    return out.reshape(b, s, d)


# speedup vs baseline: 1.0858x; 1.0858x over previous
import functools

import jax
import jax.numpy as jnp
from jax import lax
from jax.experimental import pallas as pl
from jax.experimental.pallas import tpu as pltpu
from jax.experimental.pallas import tpu_sc as plsc

NC = 2
NS = 16
NW = NC * NS
LANES = 16
CH = 32
NBUF = 2


@functools.lru_cache(maxsize=None)
def _build(nb, seq, d):
    pw = seq // NW
    n_h = pw // CH
    n_chunks = n_h * nb
    mesh = plsc.VectorSubcoreMesh(
        core_axis_name="c", subcore_axis_name="s",
        num_cores=NC, num_subcores=NS)

    @functools.partial(
        pl.kernel,
        out_type=jax.ShapeDtypeStruct((nb * seq, d), jnp.float32),
        mesh=mesh,
        scratch_types=(
            [pltpu.VMEM((pw,), jnp.int32) for _ in range(nb)]
            + [pltpu.VMEM((CH, d), jnp.float32) for _ in range(NBUF)]
            + [pltpu.VMEM((CH, d), jnp.float32)]
            + [pltpu.SemaphoreType.DMA for _ in range(2 * NBUF + 1)]
        ),
    )
    def emb(ids_hbm, wte_hbm, wpe_hbm, out_hbm, *refs):
        idx = refs[:nb]
        tok = refs[nb:nb + NBUF]
        pos_v = refs[nb + NBUF]
        sg = refs[nb + NBUF + 1:nb + NBUF + 1 + NBUF]
        ss = refs[nb + NBUF + 1 + NBUF:nb + NBUF + 1 + 2 * NBUF]
        sp = refs[nb + NBUF + 1 + 2 * NBUF]
        wid = lax.axis_index("s") * NC + lax.axis_index("c")
        pbase = wid * pw

        def parts(k):
            return k // nb, k % nb

        def add(p):
            def body(r, carry):
                for i in range(d // LANES):
                    sl = pl.ds(i * LANES, LANES)
                    tok[p][r, sl] = tok[p][r, sl] + pos_v[r, sl]
                return carry
            lax.fori_loop(0, CH, body, 0)

        def start_gather(k):
            h, b = parts(k)
            return pltpu.async_copy(
                wte_hbm.at[idx[b].at[pl.ds(h * CH, CH)]],
                tok[k % NBUF], sg[k % NBUF])

        def finish(k):
            h, b = parts(k)
            p = k % NBUF
            g[p].wait()
            add(p)
            s[p] = pltpu.async_copy(
                tok[p], out_hbm.at[pl.ds(b * seq + pbase + h * CH, CH)],
                ss[p])

        g = [None] * NBUF
        s = [None] * NBUF
        pos_pending = [None]

        for b in range(nb):
            pltpu.sync_copy(ids_hbm.at[b, pl.ds(pbase, pw)], idx[b])
        pltpu.sync_copy(wpe_hbm.at[pl.ds(pbase, CH)], pos_v)

        g[0] = start_gather(0)
        for k in range(1, n_chunks):
            p = k % NBUF
            if s[p] is not None:
                s[p].wait()
            g[p] = start_gather(k)
            if pos_pending[0] is not None:
                pos_pending[0].wait()
                pos_pending = [None]
            finish(k - 1)
            if k % nb == 0:
                hh = k // nb
                pos_pending = [pltpu.async_copy(
                    wpe_hbm.at[pl.ds(pbase + hh * CH, CH)], pos_v, sp)]
        finish(n_chunks - 1)
        for p in range(NBUF):
            if s[p] is not None:
                s[p].wait()

    return emb


def kernel(input_ids, wte, wpe):
    b, s = input_ids.shape
    d = wte.shape[1]
    emb = _build(b, s, d)
    out = emb(input_ids, wte, wpe)
    return out.reshape(b, s, d)


# speedup vs baseline: 1.1207x; 1.0322x over previous
import functools

import jax
import jax.numpy as jnp
from jax import lax
from jax.experimental import pallas as pl
from jax.experimental.pallas import tpu as pltpu
from jax.experimental.pallas import tpu_sc as plsc

NC = 2
NS = 16
NW = NC * NS
LANES = 16
CH = 32
NBUF = 2


@functools.lru_cache(maxsize=None)
def _build(nb, seq, d):
    pw = seq // NW
    n_h = pw // CH
    n_chunks = n_h * nb
    mesh = plsc.VectorSubcoreMesh(
        core_axis_name="c", subcore_axis_name="s",
        num_cores=NC, num_subcores=NS)

    @functools.partial(
        pl.kernel,
        out_type=jax.ShapeDtypeStruct((nb * seq, d), jnp.float32),
        mesh=mesh,
        scratch_types=(
            [pltpu.VMEM((pw,), jnp.int32) for _ in range(nb)]
            + [pltpu.VMEM((CH, d), jnp.float32) for _ in range(NBUF)]
            + [pltpu.VMEM((CH, d), jnp.float32)]
            + [pltpu.SemaphoreType.DMA for _ in range(2 * NBUF + 2)]
        ),
    )
    def emb(ids_hbm, wte_hbm, wpe_hbm, out_hbm, *refs):
        idx = refs[:nb]
        tok = refs[nb:nb + NBUF]
        pos_v = refs[nb + NBUF]
        sg = refs[nb + NBUF + 1:nb + NBUF + 1 + NBUF]
        ss = refs[nb + NBUF + 1 + NBUF:nb + NBUF + 1 + 2 * NBUF]
        si = refs[nb + NBUF + 1 + 2 * NBUF]
        sp = refs[nb + NBUF + 2 + 2 * NBUF]
        wid = lax.axis_index("s") * NC + lax.axis_index("c")
        pbase = wid * pw

        def parts(k):
            return k // nb, k % nb

        def add(p):
            def body(r, carry):
                for i in range(d // LANES):
                    sl = pl.ds(i * LANES, LANES)
                    tok[p][r, sl] = tok[p][r, sl] + pos_v[r, sl]
                return carry
            lax.fori_loop(0, CH, body, 0)

        def start_gather(k):
            h, b = parts(k)
            return pltpu.async_copy(
                wte_hbm.at[idx[b].at[pl.ds(h * CH, CH)]],
                tok[k % NBUF], sg[k % NBUF])

        g = [None] * NBUF
        s = [None] * NBUF
        pos_pending = [None]

        iws = [pltpu.async_copy(ids_hbm.at[b, pl.ds(pbase, pw)], idx[b], si)
               for b in range(nb)]
        pos_pending[0] = pltpu.async_copy(
            wpe_hbm.at[pl.ds(pbase, CH)], pos_v, sp)
        for iw in iws:
            iw.wait()

        g[0] = start_gather(0)
        for k in range(1, n_chunks):
            p = k % NBUF
            q = 1 - p
            if s[p] is not None:
                s[p].wait()
            g[p] = start_gather(k)
            if pos_pending[0] is not None:
                pos_pending[0].wait()
                pos_pending[0] = None
            h, b = parts(k - 1)
            g[q].wait()
            add(q)
            s[q] = pltpu.async_copy(
                tok[q], out_hbm.at[pl.ds(b * seq + pbase + h * CH, CH)],
                ss[q])
            if k % nb == 0:
                hh = k // nb
                pos_pending[0] = pltpu.async_copy(
                    wpe_hbm.at[pl.ds(pbase + hh * CH, CH)], pos_v, sp)

        p = (n_chunks - 1) % NBUF
        h, b = parts(n_chunks - 1)
        g[p].wait()
        add(p)
        s[p] = pltpu.async_copy(
            tok[p], out_hbm.at[pl.ds(b * seq + pbase + h * CH, CH)], ss[p])
        s[1 - p].wait()
        s[p].wait()

    return emb


def kernel(input_ids, wte, wpe):
    b, s = input_ids.shape
    d = wte.shape[1]
    emb = _build(b, s, d)
    out = emb(input_ids, wte, wpe)
    return out.reshape(b, s, d)
